# baseline (speedup 1.0000x reference)
import functools

import jax
import jax.numpy as jnp
from jax import lax
from jax.experimental import pallas as pl
from jax.experimental.pallas import tpu as pltpu
from jax.experimental.pallas import tpu_sc as plsc

B = 256
EMB = 16
EMBP = 128
K = 1024
HID = 32
GEN = 65536
CB = 8192
NCB = GEN // CB

_info = plsc.get_sparse_core_info()
_NC, _NS = _info.num_cores, _info.num_subcores
NW = _NC * _NS
BPW = B // NW


def _vq_kernel(z_ref, cb_ref, w1_ref, b1_ref, w2_ref, b2_ref,
               enct_ref, idx_ref, loss_ref):
    z = z_ref[...]
    cb = cb_ref[...]
    hT = lax.dot_general(cb, w1_ref[...], (((1,), (1,)), ((), ())),
                         preferred_element_type=jnp.float32) + b1_ref[...]
    hT = jnp.maximum(hT, 0.0)
    enct_ref[...] = lax.dot_general(hT, w2_ref[...], (((1,), (1,)), ((), ())),
                                    preferred_element_type=jnp.float32) \
        + b2_ref[...]
    z2 = jnp.sum(z * z, axis=1, keepdims=True)
    cb2 = jnp.sum(cb * cb, axis=1, keepdims=True)
    cross = lax.dot_general(z, cb, (((1,), (1,)), ((), ())),
                            preferred_element_type=jnp.float32)
    d = z2 - 2.0 * cross + cb2.T
    dmin = jnp.min(d, axis=1, keepdims=True)
    ii = lax.broadcasted_iota(jnp.int32, d.shape, 1)
    idx_ref[...] = jnp.min(jnp.where(d == dmin, ii, jnp.int32(K)), axis=1,
                           keepdims=True)
    loss_ref[0, 0] = 1.25 * jnp.sum(dmin) / (B * EMB)


_sc_mesh = plsc.VectorSubcoreMesh(core_axis_name="c", subcore_axis_name="s")


@functools.partial(
    pl.kernel, mesh=_sc_mesh,
    out_type=jax.ShapeDtypeStruct((B, EMBP), jnp.float32),
    scratch_types=[
        pltpu.VMEM((BPW,), jnp.int32),
        pltpu.VMEM((BPW, EMBP), jnp.float32),
        pltpu.SemaphoreType.DMA,
    ],
)
def _sc_gather(table_hbm, idx_hbm, out_hbm, idx_v, rows_v, sem):
    wid = lax.axis_index("s") * _NC + lax.axis_index("c")
    base = wid * BPW
    pltpu.sync_copy(idx_hbm.at[pl.ds(base, BPW)], idx_v)
    pltpu.async_copy(table_hbm.at[idx_v], rows_v, sem).wait()
    pltpu.sync_copy(rows_v, out_hbm.at[pl.ds(base, BPW)])


def _gen_kernel(enc_ref, wk_ref, out_ref):
    enc = enc_ref[:, :EMB]
    out_ref[...] = lax.dot_general(
        enc, wk_ref[...], (((1,), (0,)), ((), ())),
        preferred_element_type=jnp.float32)


@jax.jit
def kernel(z, codebook, W1, b1, W2, b2, Wk):
    W2p = jnp.pad(W2, ((0, EMBP - EMB), (0, 0)))
    b2p = jnp.pad(b2, (0, EMBP - EMB))
    encT, idx, loss = pl.pallas_call(
        _vq_kernel,
        out_shape=(
            jax.ShapeDtypeStruct((K, EMBP), jnp.float32),
            jax.ShapeDtypeStruct((B, 1), jnp.int32),
            jax.ShapeDtypeStruct((1, 1), jnp.float32),
        ),
        in_specs=[
            pl.BlockSpec((B, EMB), lambda: (0, 0)),
            pl.BlockSpec((K, EMB), lambda: (0, 0)),
            pl.BlockSpec((HID, EMB), lambda: (0, 0)),
            pl.BlockSpec((1, HID), lambda: (0, 0)),
            pl.BlockSpec((EMBP, HID), lambda: (0, 0)),
            pl.BlockSpec((1, EMBP), lambda: (0, 0)),
        ],
        out_specs=(
            pl.BlockSpec((K, EMBP), lambda: (0, 0)),
            pl.BlockSpec((B, 1), lambda: (0, 0)),
            pl.BlockSpec(memory_space=pltpu.SMEM),
        ),
    )(z, codebook, W1, b1.reshape(1, HID), W2p, b2p.reshape(1, EMBP))

    enc = _sc_gather(encT, idx.reshape(B))

    gen = pl.pallas_call(
        _gen_kernel,
        grid=(NCB,),
        out_shape=jax.ShapeDtypeStruct((B, GEN), jnp.float32),
        in_specs=[
            pl.BlockSpec((B, EMBP), lambda j: (0, 0)),
            pl.BlockSpec((EMB, CB), lambda j: (0, j)),
        ],
        out_specs=pl.BlockSpec((B, CB), lambda j: (0, j)),
        compiler_params=pltpu.CompilerParams(
            dimension_semantics=("arbitrary",)),
    )(enc, Wk)
    return gen, loss[0, 0]


# speedup vs baseline: 1.7281x; 1.7281x over previous
import jax
import jax.numpy as jnp
from jax import lax
from jax.experimental import pallas as pl
from jax.experimental.pallas import tpu as pltpu

B = 256
EMB = 16
K = 1024
HID = 32
GEN = 65536
CB = 8192
NCB = GEN // CB


def _fused_kernel(z_ref, cb_ref, w1_ref, b1_ref, w2_ref, b2_ref, wk_ref,
                  out_ref, loss_ref, enc_s):
    j = pl.program_id(0)

    @pl.when(j == 0)
    def _vq_trunk():
        z = z_ref[...]
        cb = cb_ref[...]
        hT = lax.dot_general(cb, w1_ref[...], (((1,), (1,)), ((), ())),
                             preferred_element_type=jnp.float32) + b1_ref[...]
        hT = jnp.maximum(hT, 0.0)
        encT = lax.dot_general(hT, w2_ref[...], (((1,), (1,)), ((), ())),
                               preferred_element_type=jnp.float32) \
            + b2_ref[...]
        z2 = jnp.sum(z * z, axis=1, keepdims=True)
        cb2 = jnp.sum(cb * cb, axis=1, keepdims=True)
        cross = lax.dot_general(z, cb, (((1,), (1,)), ((), ())),
                                preferred_element_type=jnp.float32)
        d = z2 - 2.0 * cross + cb2.T
        dmin = jnp.min(d, axis=1, keepdims=True)
        ii = lax.broadcasted_iota(jnp.int32, d.shape, 1)
        idx = jnp.min(jnp.where(d == dmin, ii, jnp.int32(K)), axis=1,
                      keepdims=True)
        loss_ref[0, 0] = 1.25 * jnp.sum(dmin) / (B * EMB)
        onehot = (ii == idx).astype(jnp.float32)
        enc_s[...] = lax.dot_general(onehot, encT, (((1,), (0,)), ((), ())),
                                     preferred_element_type=jnp.float32)

    out_ref[...] = lax.dot_general(
        enc_s[...], wk_ref[...], (((1,), (0,)), ((), ())),
        preferred_element_type=jnp.float32)


@jax.jit
def kernel(z, codebook, W1, b1, W2, b2, Wk):
    gen, loss = pl.pallas_call(
        _fused_kernel,
        grid=(NCB,),
        out_shape=(
            jax.ShapeDtypeStruct((B, GEN), jnp.float32),
            jax.ShapeDtypeStruct((1, 1), jnp.float32),
        ),
        in_specs=[
            pl.BlockSpec((B, EMB), lambda j: (0, 0)),
            pl.BlockSpec((K, EMB), lambda j: (0, 0)),
            pl.BlockSpec((HID, EMB), lambda j: (0, 0)),
            pl.BlockSpec((1, HID), lambda j: (0, 0)),
            pl.BlockSpec((EMB, HID), lambda j: (0, 0)),
            pl.BlockSpec((1, EMB), lambda j: (0, 0)),
            pl.BlockSpec((EMB, CB), lambda j: (0, j)),
        ],
        out_specs=(
            pl.BlockSpec((B, CB), lambda j: (0, j)),
            pl.BlockSpec(memory_space=pltpu.SMEM),
        ),
        scratch_shapes=[pltpu.VMEM((B, EMB), jnp.float32)],
        compiler_params=pltpu.CompilerParams(
            dimension_semantics=("arbitrary",)),
    )(z, codebook, W1, b1.reshape(1, HID), W2, b2.reshape(1, EMB), Wk)
    return gen, loss[0, 0]
